# baseline (speedup 1.0000x reference)
import jax
import jax.numpy as jnp
from jax.experimental import pallas as pl
from jax.experimental.pallas import tpu as pltpu

_ROWS = 100000
_BLOCK = 4000


def _body(x0_ref, x1_ref, x2_ref, x3_ref, o0_ref, o1_ref):
    x0 = x0_ref[...]
    x1 = x1_ref[...]
    x2 = x2_ref[...]
    x3 = x3_ref[...]
    o0_ref[...] = jnp.concatenate(
        [x0[:, 0:64], x1[:, 0:96], x2[:, 64:128], x3[:, 0:16]], axis=1)
    o1_ref[...] = jnp.concatenate(
        [x0[:, 64:128], x1[:, 96:128], x2[:, 0:64], x3[:, 16:96]], axis=1)


@jax.jit
def kernel(x):
    return pl.pallas_call(
        _body,
        grid=(_ROWS // _BLOCK,),
        in_specs=[
            pl.BlockSpec((_BLOCK, 128), lambda i: (i, 0)),
            pl.BlockSpec((_BLOCK, 128), lambda i: (i, 1)),
            pl.BlockSpec((_BLOCK, 128), lambda i: (i, 2)),
            pl.BlockSpec((_BLOCK, 128), lambda i: (i, 3)),
        ],
        out_specs=(
            pl.BlockSpec((_BLOCK, 240), lambda i: (i, 0)),
            pl.BlockSpec((_BLOCK, 240), lambda i: (i, 0)),
        ),
        out_shape=(
            jax.ShapeDtypeStruct((_ROWS, 240), jnp.float32),
            jax.ShapeDtypeStruct((_ROWS, 240), jnp.float32),
        ),
        compiler_params=pltpu.CompilerParams(
            dimension_semantics=("parallel",),
        ),
    )(x, x, x, x)
